# baseline (speedup 1.0000x reference)
import functools

import jax
import jax.numpy as jnp
import numpy as np
from jax import lax
from jax.experimental import pallas as pl
from jax.experimental.pallas import tpu as pltpu
from jax.experimental.pallas import tpu_sc as plsc

N = 10000
D = 128
E = 320000
H1, C1 = 8, 8
C2 = 16

NC, NS, LANES = 2, 16, 16
NW = NC * NS

NP = 10240
ROWS_PER_TILE = NP // NS
WB_ROWS = 128
WB_CHUNKS = ROWS_PER_TILE // WB_ROWS
E_PAD = 327680
IDX_COLS = 128
EROWS = E_PAD // IDX_COLS
ROWS_PER_W = EROWS // NW
CH = 256
IDXR = CH // IDX_COLS
NCHUNK = ROWS_PER_W // IDXR
SUPERS = NCHUNK // 2


def _make_edge_kernel(row_w: int, gather_w: bool, bf16_feat: bool = False):
    parts = row_w // LANES
    mesh = plsc.VectorSubcoreMesh(core_axis_name="c", subcore_axis_name="s")

    per_set = [
        pltpu.VMEM((IDXR, IDX_COLS), jnp.int32),
        pltpu.VMEM((IDXR, IDX_COLS), jnp.int32),
        pltpu.VMEM((IDXR, IDX_COLS), jnp.int32),
        pltpu.VMEM((CH, 16), jnp.float32),
        pltpu.VMEM((CH, 16), jnp.float32),
        pltpu.VMEM((CH, 16), jnp.float32),
        pltpu.VMEM((CH, row_w), jnp.float32),
    ]
    if bf16_feat:
        per_set.append(pltpu.VMEM((CH, row_w), jnp.bfloat16))
    nset = len(per_set)
    scratch = per_set + per_set + [
        pltpu.VMEM_SHARED((NP, row_w), jnp.float32),
        pltpu.VMEM_SHARED((NP, 16), jnp.float32),
        pltpu.SemaphoreType.DMA,
        pltpu.SemaphoreType.DMA,
        pltpu.SemaphoreType.DMA,
        pltpu.SemaphoreType.DMA,
        pltpu.SemaphoreType.DMA,
        pltpu.SemaphoreType.DMA,
    ]

    @functools.partial(
        pl.kernel,
        out_type=(
            jax.ShapeDtypeStruct((NC, NP, row_w), jnp.float32),
            jax.ShapeDtypeStruct((NC, NP, 16), jnp.float32),
        ),
        mesh=mesh,
        scratch_types=scratch,
        compiler_params=pltpu.CompilerParams(
            needs_layout_passes=False, use_tc_tiling_on_sc=False),
    )
    def edge_kernel(src_hbm, dst_hbm, h_hbm, s_hbm, d_hbm, num_out, den_out,
                    *bufs):
        sets = [bufs[:nset], bufs[nset:2 * nset]]
        num_sh, den_sh = bufs[2 * nset:2 * nset + 2]
        gsem = bufs[2 * nset + 2:2 * nset + 4]
        isem = bufs[2 * nset + 4:2 * nset + 6]
        ssem = bufs[2 * nset + 6:2 * nset + 8]
        zb_v = sets[1][6].at[pl.ds(0, WB_ROWS)]
        zs_v = sets[1][5].at[pl.ds(0, WB_ROWS)]
        c = lax.axis_index("c")
        s = lax.axis_index("s")
        wid = c * NS + s
        zero16 = jnp.zeros((16,), jnp.float32)

        def zrow(r, _):
            for p in range(parts):
                zb_v[r, pl.ds(p * LANES, LANES)] = zero16
            zs_v[r, :] = zero16
            return 0

        lax.fori_loop(0, WB_ROWS, zrow, 0)
        r0 = s * ROWS_PER_TILE

        def zcp(k, _):
            rk = r0 + k * WB_ROWS
            pltpu.sync_copy(zb_v, num_sh.at[pl.ds(rk, WB_ROWS)])
            pltpu.sync_copy(zs_v, den_sh.at[pl.ds(rk, WB_ROWS)])
            return 0

        lane = lax.iota(jnp.int32, 16)
        head_pat = lane >> 3
        quad_pat = lane >> 2

        def idx_issue(chunk_i, st, sem):
            er0 = wid * ROWS_PER_W + chunk_i * IDXR
            pltpu.async_copy(src_hbm.at[pl.ds(er0, IDXR)], st[0], sem)
            pltpu.async_copy(dst_hbm.at[pl.ds(er0, IDXR)], st[1], sem)

        def idx_drain(st, sem):
            pltpu.make_async_copy(
                src_hbm.at[pl.ds(0, IDXR)], st[0], sem).wait()
            pltpu.make_async_copy(
                dst_hbm.at[pl.ds(0, IDXR)], st[1], sem).wait()

        def gat_issue(st, sem):
            for j in range(IDXR):
                sl = pl.ds(j * IDX_COLS, IDX_COLS)
                pltpu.async_copy(s_hbm.at[st[0].at[j]], st[3].at[sl], sem)
                pltpu.async_copy(d_hbm.at[st[1].at[j]], st[4].at[sl], sem)
                pltpu.async_copy(
                    h_hbm.at[st[0].at[j]],
                    (st[7] if bf16_feat else st[6]).at[sl], sem)

        def gat_drain(st, sem):
            for j in range(IDXR):
                sl = pl.ds(j * IDX_COLS, IDX_COLS)
                pltpu.make_async_copy(
                    s_hbm.at[st[0].at[j]], st[3].at[sl], sem).wait()
                pltpu.make_async_copy(
                    d_hbm.at[st[1].at[j]], st[4].at[sl], sem).wait()
                pltpu.make_async_copy(
                    h_hbm.at[st[0].at[j]],
                    (st[7] if bf16_feat else st[6]).at[sl], sem).wait()

        def scat_issue(st, sem):
            for j in range(IDXR):
                sl = pl.ds(j * IDX_COLS, IDX_COLS)
                pltpu.async_copy(
                    st[6].at[sl], num_sh.at[st[2].at[j]], sem, add=True)
                pltpu.async_copy(
                    st[5].at[sl], den_sh.at[st[2].at[j]], sem, add=True)

        def scat_drain(st, sem):
            for j in range(IDXR):
                sl = pl.ds(j * IDX_COLS, IDX_COLS)
                pltpu.make_async_copy(
                    st[6].at[sl], num_sh.at[st[2].at[j]], sem).wait()
                pltpu.make_async_copy(
                    st[5].at[sl], den_sh.at[st[2].at[j]], sem).wait()

        def compute(st):
            s_v, d_v, w_v, m_v = st[3], st[4], st[5], st[6]
            if bf16_feat:
                mb_v = st[7]

                def body(e2, _):
                    for u in range(2):
                        e = e2 * 2 + u
                        v = s_v[e, :] + d_v[e, :]
                        v = jnp.where(v > 0.0, v, 0.2 * v)
                        w = jnp.exp(v)
                        w_v[e, :] = w
                        for t in range(row_w // 32):
                            xb = mb_v[e, pl.ds(32 * t, 32)]
                            a, b = plsc.unpack(
                                xb, format=plsc.PackFormat.INTERLEAVED)
                            wv = w.at[quad_pat + 4 * t].get(
                                mode="promise_in_bounds")
                            m_v[e, pl.ds(32 * t, 16)] = a * wv
                            m_v[e, pl.ds(32 * t + 16, 16)] = b * wv
                    return 0
            elif gather_w:
                def body(e2, _):
                    for u in range(2):
                        e = e2 * 2 + u
                        v = s_v[e, :] + d_v[e, :]
                        v = jnp.where(v > 0.0, v, 0.2 * v)
                        w = jnp.exp(v)
                        w_v[e, :] = w
                        for q in range(parts):
                            wv = w.at[head_pat + 2 * q].get(
                                mode="promise_in_bounds")
                            sl = pl.ds(q * LANES, LANES)
                            m_v[e, sl] = m_v[e, sl] * wv
                    return 0
            else:
                def body(e2, _):
                    for u in range(2):
                        e = e2 * 2 + u
                        v = s_v[e, :] + d_v[e, :]
                        v = jnp.where(v > 0.0, v, 0.2 * v)
                        w = jnp.exp(v)
                        w_v[e, :] = w
                        m_v[e, :] = m_v[e, :] * w
                    return 0

            lax.fori_loop(0, CH // 2, body, 0)

        idx_issue(0, sets[0], isem[0])
        idx_drain(sets[0], isem[0])
        idx_issue(1, sets[1], isem[1])
        gat_issue(sets[0], gsem[0])
        lax.fori_loop(0, WB_CHUNKS, zcp, 0)
        idx_drain(sets[1], isem[1])
        plsc.subcore_barrier()

        def super_body(s2, _):
            for b in range(2):
                i = s2 * 2 + b
                st = sets[b]
                ot = sets[1 - b]
                if b == 1:
                    scat_drain(ot, ssem[0])
                else:
                    @pl.when(s2 >= 1)
                    def _(ot=ot):
                        scat_drain(ot, ssem[1])
                if b == 1:
                    @pl.when(s2 < SUPERS - 1)
                    def _(ot=ot):
                        idx_drain(ot, isem[0])
                else:
                    @pl.when(s2 >= 1)
                    def _(ot=ot):
                        idx_drain(ot, isem[1])
                if b == 1:
                    @pl.when(s2 < SUPERS - 1)
                    def _(ot=ot):
                        gat_issue(ot, gsem[0])
                else:
                    gat_issue(ot, gsem[1])
                gat_drain(st, gsem[b])
                for j in range(IDXR):
                    for p in range(IDX_COLS // LANES):
                        sl = pl.ds(p * LANES, LANES)
                        st[2][j, sl] = st[1][j, sl]
                @pl.when(s2 < SUPERS - 1)
                def _(st=st, i=i, b=b):
                    idx_issue(i + 2, st, isem[b])
                compute(st)
                scat_issue(st, ssem[b])
            return 0

        lax.fori_loop(0, SUPERS, super_body, 0)
        scat_drain(sets[1], ssem[1])
        plsc.subcore_barrier()

        def wb(k, _):
            rk = r0 + k * WB_ROWS
            pltpu.sync_copy(num_sh.at[pl.ds(rk, WB_ROWS)], zb_v)
            pltpu.sync_copy(zb_v, num_out.at[c, pl.ds(rk, WB_ROWS)])
            pltpu.sync_copy(den_sh.at[pl.ds(rk, WB_ROWS)], zs_v)
            pltpu.sync_copy(zs_v, den_out.at[c, pl.ds(rk, WB_ROWS)])
            return 0

        lax.fori_loop(0, WB_CHUNKS, wb, 0)

    return edge_kernel


_edge1 = _make_edge_kernel(H1 * C1, True, bf16_feat=True)
_edge2 = _make_edge_kernel(C2, False)

_PCOL = np.concatenate(
    [np.concatenate([np.arange(32 * t, 32 * t + 32, 2),
                     np.arange(32 * t + 1, 32 * t + 32, 2)])
     for t in range(2)])


def _tc_layer1(x_pad, W1, As1, Ad1):
    def body(x_ref, w_ref, as_ref, ad_ref, h_ref, s_ref, d_ref):
        h = jnp.dot(x_ref[...], w_ref[...],
                    preferred_element_type=jnp.float32)
        h_ref[...] = h.astype(jnp.bfloat16)
        s_ref[...] = jnp.dot(h, as_ref[...],
                             preferred_element_type=jnp.float32)
        d_ref[...] = jnp.dot(h, ad_ref[...],
                             preferred_element_type=jnp.float32)

    return pl.pallas_call(
        body,
        out_shape=(
            jax.ShapeDtypeStruct((NP, H1 * C1), jnp.bfloat16),
            jax.ShapeDtypeStruct((NP, 16), jnp.float32),
            jax.ShapeDtypeStruct((NP, 16), jnp.float32),
        ),
    )(x_pad, W1, As1, Ad1)


def _tc_mid(num1, den1, b1, W2, As2, Ad2, Kmat):
    def body(n_ref, d_ref, b_ref, w_ref, as_ref, ad_ref, k_ref,
             h_ref, s_ref, dd_ref):
        num = n_ref[0] + n_ref[1]
        den = d_ref[0] + d_ref[1]
        dexp = jnp.dot(den, k_ref[...],
                       preferred_element_type=jnp.float32) + 1e-16
        z = num / dexp + b_ref[...]
        z = jnp.where(z > 0.0, z, jnp.exp(z) - 1.0)
        h2 = jnp.dot(z, w_ref[...], preferred_element_type=jnp.float32)
        h_ref[...] = h2
        s_ref[...] = jnp.dot(h2, as_ref[...],
                             preferred_element_type=jnp.float32)
        dd_ref[...] = jnp.dot(h2, ad_ref[...],
                              preferred_element_type=jnp.float32)

    return pl.pallas_call(
        body,
        out_shape=(
            jax.ShapeDtypeStruct((NP, C2), jnp.float32),
            jax.ShapeDtypeStruct((NP, 16), jnp.float32),
            jax.ShapeDtypeStruct((NP, 16), jnp.float32),
        ),
    )(num1, den1, b1, W2, As2, Ad2, Kmat)


def _tc_final(num2, den2, b2):
    def body(n_ref, d_ref, b_ref, o_ref):
        o = (n_ref[0] + n_ref[1]) / (d_ref[0] + d_ref[1] + 1e-16)
        o = o + b_ref[...]
        m = jnp.max(o, axis=1, keepdims=True)
        lse = m + jnp.log(jnp.sum(jnp.exp(o - m), axis=1, keepdims=True))
        o_ref[...] = o - lse

    return pl.pallas_call(
        body,
        out_shape=jax.ShapeDtypeStruct((NP, C2), jnp.float32),
    )(num2, den2, b2)


def kernel(x, edge_index, W1, a_src1, a_dst1, b1, W2, a_src2, a_dst2, b2):
    x_pad = jnp.zeros((NP, D), jnp.float32).at[:N].set(x)
    src = jnp.concatenate(
        [edge_index[0], jnp.full((E_PAD - E,), N, edge_index.dtype)]
    ).reshape(EROWS, IDX_COLS).astype(jnp.int32)
    dst = jnp.concatenate(
        [edge_index[1], jnp.full((E_PAD - E,), N, edge_index.dtype)]
    ).reshape(EROWS, IDX_COLS).astype(jnp.int32)

    eye8 = jnp.eye(8, dtype=jnp.float32)
    Bs = (a_src1[:, :, None] * eye8[:, None, :]).reshape(H1 * C1, H1)
    Bd = (a_dst1[:, :, None] * eye8[:, None, :]).reshape(H1 * C1, H1)
    As1 = jnp.concatenate([Bs, Bs], axis=1)
    Ad1 = jnp.concatenate([Bd, Bd], axis=1)
    Kmat = jnp.concatenate(
        [jnp.repeat(eye8, 8, axis=1), jnp.zeros((8, 64), jnp.float32)],
        axis=0)
    Kmat = Kmat[:, _PCOL]
    W2 = W2[_PCOL, :]
    As2 = jnp.tile(a_src2.reshape(C2, 1), (1, 16))
    Ad2 = jnp.tile(a_dst2.reshape(C2, 1), (1, 16))
    b1r = b1[_PCOL].reshape(1, H1 * C1)
    b2r = b2.reshape(1, C2)

    h1, s1, d1 = _tc_layer1(x_pad, W1, As1, Ad1)
    num1, den1 = _edge1(src, dst, h1, s1, d1)
    h2, s2, d2 = _tc_mid(num1, den1, b1r, W2, As2, Ad2, Kmat)
    num2, den2 = _edge2(src, dst, h2, s2, d2)
    out = _tc_final(num2, den2, b2r)
    return out[:N]


# speedup vs baseline: 1.0162x; 1.0162x over previous
import functools

import jax
import jax.numpy as jnp
from jax import lax
from jax.experimental import pallas as pl
from jax.experimental.pallas import tpu as pltpu
from jax.experimental.pallas import tpu_sc as plsc

N = 10000
D = 128
E = 320000
H1, C1 = 8, 8
C2 = 16

NC, NS, LANES = 2, 16, 16
NW = NC * NS

NP = 10240
ROWS_PER_TILE = NP // NS
WB_ROWS = 128
WB_CHUNKS = ROWS_PER_TILE // WB_ROWS
E_PAD = 327680
IDX_COLS = 128
EROWS = E_PAD // IDX_COLS
ROWS_PER_W = EROWS // NW
CH = 256
IDXR = CH // IDX_COLS
NCHUNK = ROWS_PER_W // IDXR
SUPERS = NCHUNK // 2


def _make_edge_kernel(row_w: int, gather_w: bool):
    parts = row_w // LANES
    mesh = plsc.VectorSubcoreMesh(core_axis_name="c", subcore_axis_name="s")

    per_set = [
        pltpu.VMEM((IDXR, IDX_COLS), jnp.int32),
        pltpu.VMEM((IDXR, IDX_COLS), jnp.int32),
        pltpu.VMEM((IDXR, IDX_COLS), jnp.int32),
        pltpu.VMEM((CH, 16), jnp.float32),
        pltpu.VMEM((CH, 16), jnp.float32),
        pltpu.VMEM((CH, 16), jnp.float32),
        pltpu.VMEM((CH, row_w), jnp.float32),
    ]
    nset = len(per_set)
    scratch = per_set + per_set + [
        pltpu.VMEM((WB_ROWS, row_w), jnp.float32),
        pltpu.VMEM((WB_ROWS, 16), jnp.float32),
        pltpu.VMEM_SHARED((NP, row_w), jnp.float32),
        pltpu.VMEM_SHARED((NP, 16), jnp.float32),
        pltpu.SemaphoreType.DMA,
        pltpu.SemaphoreType.DMA,
        pltpu.SemaphoreType.DMA,
        pltpu.SemaphoreType.DMA,
        pltpu.SemaphoreType.DMA,
        pltpu.SemaphoreType.DMA,
    ]

    @functools.partial(
        pl.kernel,
        out_type=(
            jax.ShapeDtypeStruct((NC, NP, row_w), jnp.float32),
            jax.ShapeDtypeStruct((NC, NP, 16), jnp.float32),
        ),
        mesh=mesh,
        scratch_types=scratch,
        compiler_params=pltpu.CompilerParams(
            needs_layout_passes=False, use_tc_tiling_on_sc=False),
    )
    def edge_kernel(src_hbm, dst_hbm, h_hbm, s_hbm, d_hbm, num_out, den_out,
                    *bufs):
        sets = [bufs[:nset], bufs[nset:2 * nset]]
        zb_v, zs_v, num_sh, den_sh = bufs[2 * nset:2 * nset + 4]
        gsem = bufs[2 * nset + 4:2 * nset + 6]
        isem = bufs[2 * nset + 6:2 * nset + 8]
        ssem = bufs[2 * nset + 8:2 * nset + 10]
        c = lax.axis_index("c")
        s = lax.axis_index("s")
        wid = c * NS + s
        zero16 = jnp.zeros((16,), jnp.float32)

        def zrow(r, _):
            for p in range(parts):
                zb_v[r, pl.ds(p * LANES, LANES)] = zero16
            zs_v[r, :] = zero16
            return 0

        lax.fori_loop(0, WB_ROWS, zrow, 0)
        r0 = s * ROWS_PER_TILE

        def zcp(k, _):
            rk = r0 + k * WB_ROWS
            pltpu.sync_copy(zb_v, num_sh.at[pl.ds(rk, WB_ROWS)])
            pltpu.sync_copy(zs_v, den_sh.at[pl.ds(rk, WB_ROWS)])
            return 0

        lane = lax.iota(jnp.int32, 16)
        head_pat = lane >> 3

        def idx_issue(chunk_i, st, sem):
            er0 = wid * ROWS_PER_W + chunk_i * IDXR
            pltpu.async_copy(src_hbm.at[pl.ds(er0, IDXR)], st[0], sem)
            pltpu.async_copy(dst_hbm.at[pl.ds(er0, IDXR)], st[1], sem)

        def idx_drain(st, sem):
            pltpu.make_async_copy(
                src_hbm.at[pl.ds(0, IDXR)], st[0], sem).wait()
            pltpu.make_async_copy(
                dst_hbm.at[pl.ds(0, IDXR)], st[1], sem).wait()

        def gat_issue(st, sem):
            for j in range(IDXR):
                sl = pl.ds(j * IDX_COLS, IDX_COLS)
                pltpu.async_copy(s_hbm.at[st[0].at[j]], st[3].at[sl], sem)
                pltpu.async_copy(d_hbm.at[st[1].at[j]], st[4].at[sl], sem)
                pltpu.async_copy(h_hbm.at[st[0].at[j]], st[6].at[sl], sem)

        def gat_drain(st, sem):
            for j in range(IDXR):
                sl = pl.ds(j * IDX_COLS, IDX_COLS)
                pltpu.make_async_copy(
                    s_hbm.at[st[0].at[j]], st[3].at[sl], sem).wait()
                pltpu.make_async_copy(
                    d_hbm.at[st[1].at[j]], st[4].at[sl], sem).wait()
                pltpu.make_async_copy(
                    h_hbm.at[st[0].at[j]], st[6].at[sl], sem).wait()

        def scat_issue(st, sem):
            for j in range(IDXR):
                sl = pl.ds(j * IDX_COLS, IDX_COLS)
                pltpu.async_copy(
                    st[6].at[sl], num_sh.at[st[2].at[j]], sem, add=True)
                pltpu.async_copy(
                    st[5].at[sl], den_sh.at[st[2].at[j]], sem, add=True)

        def scat_drain(st, sem):
            for j in range(IDXR):
                sl = pl.ds(j * IDX_COLS, IDX_COLS)
                pltpu.make_async_copy(
                    st[6].at[sl], num_sh.at[st[2].at[j]], sem).wait()
                pltpu.make_async_copy(
                    st[5].at[sl], den_sh.at[st[2].at[j]], sem).wait()

        def compute(st):
            s_v, d_v, w_v, m_v = st[3], st[4], st[5], st[6]
            if gather_w:
                def body(e2, _):
                    for u in range(2):
                        e = e2 * 2 + u
                        v = s_v[e, :] + d_v[e, :]
                        v = jnp.where(v > 0.0, v, 0.2 * v)
                        w = jnp.exp(v)
                        w_v[e, :] = w
                        for q in range(parts):
                            wv = w.at[head_pat + 2 * q].get(
                                mode="promise_in_bounds")
                            sl = pl.ds(q * LANES, LANES)
                            m_v[e, sl] = m_v[e, sl] * wv
                    return 0
            else:
                def body(e2, _):
                    for u in range(2):
                        e = e2 * 2 + u
                        v = s_v[e, :] + d_v[e, :]
                        v = jnp.where(v > 0.0, v, 0.2 * v)
                        w = jnp.exp(v)
                        w_v[e, :] = w
                        m_v[e, :] = m_v[e, :] * w
                    return 0

            lax.fori_loop(0, CH // 2, body, 0)

        idx_issue(0, sets[0], isem[0])
        idx_drain(sets[0], isem[0])
        idx_issue(1, sets[1], isem[1])
        gat_issue(sets[0], gsem[0])
        lax.fori_loop(0, WB_CHUNKS, zcp, 0)
        idx_drain(sets[1], isem[1])
        plsc.subcore_barrier()

        def super_body(s2, _):
            for b in range(2):
                i = s2 * 2 + b
                st = sets[b]
                ot = sets[1 - b]
                if b == 1:
                    scat_drain(ot, ssem[0])
                else:
                    @pl.when(s2 >= 1)
                    def _(ot=ot):
                        scat_drain(ot, ssem[1])
                if b == 1:
                    @pl.when(s2 < SUPERS - 1)
                    def _(ot=ot):
                        idx_drain(ot, isem[0])
                else:
                    @pl.when(s2 >= 1)
                    def _(ot=ot):
                        idx_drain(ot, isem[1])
                if b == 1:
                    @pl.when(s2 < SUPERS - 1)
                    def _(ot=ot):
                        gat_issue(ot, gsem[0])
                else:
                    gat_issue(ot, gsem[1])
                gat_drain(st, gsem[b])
                for j in range(IDXR):
                    for p in range(IDX_COLS // LANES):
                        sl = pl.ds(p * LANES, LANES)
                        st[2][j, sl] = st[1][j, sl]
                @pl.when(s2 < SUPERS - 1)
                def _(st=st, i=i, b=b):
                    idx_issue(i + 2, st, isem[b])
                compute(st)
                scat_issue(st, ssem[b])
            return 0

        lax.fori_loop(0, SUPERS, super_body, 0)
        scat_drain(sets[1], ssem[1])
        plsc.subcore_barrier()

        def wb(k, _):
            rk = r0 + k * WB_ROWS
            pltpu.sync_copy(num_sh.at[pl.ds(rk, WB_ROWS)], zb_v)
            pltpu.sync_copy(zb_v, num_out.at[c, pl.ds(rk, WB_ROWS)])
            pltpu.sync_copy(den_sh.at[pl.ds(rk, WB_ROWS)], zs_v)
            pltpu.sync_copy(zs_v, den_out.at[c, pl.ds(rk, WB_ROWS)])
            return 0

        lax.fori_loop(0, WB_CHUNKS, wb, 0)

    return edge_kernel


_edge1 = _make_edge_kernel(H1 * C1, True)
_edge2 = _make_edge_kernel(C2, False)


def _tc_layer1(x_pad, W1, As1, Ad1):
    def body(x_ref, w_ref, as_ref, ad_ref, h_ref, s_ref, d_ref):
        h = jnp.dot(x_ref[...], w_ref[...],
                    preferred_element_type=jnp.float32)
        h_ref[...] = h
        s_ref[...] = jnp.dot(h, as_ref[...],
                             preferred_element_type=jnp.float32)
        d_ref[...] = jnp.dot(h, ad_ref[...],
                             preferred_element_type=jnp.float32)

    return pl.pallas_call(
        body,
        out_shape=(
            jax.ShapeDtypeStruct((NP, H1 * C1), jnp.float32),
            jax.ShapeDtypeStruct((NP, 16), jnp.float32),
            jax.ShapeDtypeStruct((NP, 16), jnp.float32),
        ),
    )(x_pad, W1, As1, Ad1)


def _tc_mid(num1, den1, b1, W2, As2, Ad2, Kmat):
    def body(n_ref, d_ref, b_ref, w_ref, as_ref, ad_ref, k_ref,
             h_ref, s_ref, dd_ref):
        num = n_ref[0] + n_ref[1]
        den = d_ref[0] + d_ref[1]
        dexp = jnp.dot(den, k_ref[...],
                       preferred_element_type=jnp.float32) + 1e-16
        z = num / dexp + b_ref[...]
        z = jnp.where(z > 0.0, z, jnp.exp(z) - 1.0)
        h2 = jnp.dot(z, w_ref[...], preferred_element_type=jnp.float32)
        h_ref[...] = h2
        s_ref[...] = jnp.dot(h2, as_ref[...],
                             preferred_element_type=jnp.float32)
        dd_ref[...] = jnp.dot(h2, ad_ref[...],
                              preferred_element_type=jnp.float32)

    return pl.pallas_call(
        body,
        out_shape=(
            jax.ShapeDtypeStruct((NP, C2), jnp.float32),
            jax.ShapeDtypeStruct((NP, 16), jnp.float32),
            jax.ShapeDtypeStruct((NP, 16), jnp.float32),
        ),
    )(num1, den1, b1, W2, As2, Ad2, Kmat)


def _tc_final(num2, den2, b2):
    def body(n_ref, d_ref, b_ref, o_ref):
        o = (n_ref[0] + n_ref[1]) / (d_ref[0] + d_ref[1] + 1e-16)
        o = o + b_ref[...]
        m = jnp.max(o, axis=1, keepdims=True)
        lse = m + jnp.log(jnp.sum(jnp.exp(o - m), axis=1, keepdims=True))
        o_ref[...] = o - lse

    return pl.pallas_call(
        body,
        out_shape=jax.ShapeDtypeStruct((NP, C2), jnp.float32),
    )(num2, den2, b2)


def kernel(x, edge_index, W1, a_src1, a_dst1, b1, W2, a_src2, a_dst2, b2):
    x_pad = jnp.zeros((NP, D), jnp.float32).at[:N].set(x)
    src = jnp.concatenate(
        [edge_index[0], jnp.full((E_PAD - E,), N, edge_index.dtype)]
    ).reshape(EROWS, IDX_COLS).astype(jnp.int32)
    dst = jnp.concatenate(
        [edge_index[1], jnp.full((E_PAD - E,), N, edge_index.dtype)]
    ).reshape(EROWS, IDX_COLS).astype(jnp.int32)

    eye8 = jnp.eye(8, dtype=jnp.float32)
    Bs = (a_src1[:, :, None] * eye8[:, None, :]).reshape(H1 * C1, H1)
    Bd = (a_dst1[:, :, None] * eye8[:, None, :]).reshape(H1 * C1, H1)
    As1 = jnp.concatenate([Bs, Bs], axis=1)
    Ad1 = jnp.concatenate([Bd, Bd], axis=1)
    Kmat = jnp.concatenate(
        [jnp.repeat(eye8, 8, axis=1), jnp.zeros((8, 64), jnp.float32)],
        axis=0)
    As2 = jnp.tile(a_src2.reshape(C2, 1), (1, 16))
    Ad2 = jnp.tile(a_dst2.reshape(C2, 1), (1, 16))
    b1r = b1.reshape(1, H1 * C1)
    b2r = b2.reshape(1, C2)

    h1, s1, d1 = _tc_layer1(x_pad, W1, As1, Ad1)
    num1, den1 = _edge1(src, dst, h1, s1, d1)
    h2, s2, d2 = _tc_mid(num1, den1, b1r, W2, As2, Ad2, Kmat)
    num2, den2 = _edge2(src, dst, h2, s2, d2)
    out = _tc_final(num2, den2, b2r)
    return out[:N]
